# Initial kernel scaffold; baseline (speedup 1.0000x reference)
import jax
import jax.numpy as jnp
from jax.experimental import pallas as pl


def kernel(news_batch, news_id, news_repr_table, news_embedding_table):
    raise NotImplementedError("write your pallas kernel here")


# speedup vs baseline: 1.1731x; 1.1731x over previous
import functools

import jax
import jax.numpy as jnp
from jax import lax
from jax.experimental import pallas as pl
from jax.experimental.pallas import tpu as pltpu
from jax.experimental.pallas import tpu_sc as plsc

_NC = 2
_NS = 16
_NW = _NC * _NS

_CHUNK = 64


@functools.lru_cache(maxsize=None)
def _make_gather(N, D_emb, D_repr):
    per_w = N // _NW
    n_chunks = per_w // _CHUNK
    assert per_w % _CHUNK == 0 and N % _NW == 0
    mesh = plsc.VectorSubcoreMesh(core_axis_name="c", subcore_axis_name="s")

    @functools.partial(
        pl.kernel,
        mesh=mesh,
        compiler_params=pltpu.CompilerParams(use_tc_tiling_on_sc=False),
        out_type=(
            jax.ShapeDtypeStruct((N, D_emb), jnp.float32),
            jax.ShapeDtypeStruct((N, D_repr), jnp.float32),
        ),
        scratch_types=[
            pltpu.VMEM((per_w,), jnp.int32),
            pltpu.VMEM((_CHUNK, D_emb), jnp.float32),
            pltpu.VMEM((_CHUNK, D_repr), jnp.float32),
            pltpu.SemaphoreType.DMA,
            pltpu.SemaphoreType.DMA,
        ],
    )
    def gather_kernel(idx_hbm, emb_tab, repr_tab, out_emb, out_repr,
                      idx_v, emb_v, repr_v, sem_e, sem_r):
        wid = lax.axis_index("s") * _NC + lax.axis_index("c")
        base = wid * per_w
        pltpu.sync_copy(idx_hbm.at[pl.ds(base, per_w)], idx_v)

        def body(c, carry):
            off = c * _CHUNK
            ge = pltpu.async_copy(
                emb_tab.at[idx_v.at[pl.ds(off, _CHUNK)]], emb_v, sem_e)
            gr = pltpu.async_copy(
                repr_tab.at[idx_v.at[pl.ds(off, _CHUNK)]], repr_v, sem_r)
            ge.wait()
            gr.wait()
            pltpu.sync_copy(emb_v, out_emb.at[pl.ds(base + off, _CHUNK)])
            pltpu.sync_copy(repr_v, out_repr.at[pl.ds(base + off, _CHUNK)])
            return carry

        lax.fori_loop(0, n_chunks, body, 0)

    return gather_kernel


def kernel(news_batch, news_id, news_repr_table, news_embedding_table):
    B, H = news_id.shape
    N = B * H
    hidden = news_repr_table.shape[1]
    level = news_embedding_table.shape[1] // hidden
    idx = news_id.reshape(N).astype(jnp.int32)
    gather = _make_gather(N, news_embedding_table.shape[1], hidden)
    out_emb, out_repr = gather(idx, news_embedding_table, news_repr_table)
    news_embedding = out_emb.reshape(B, H, level, hidden)
    news_repr = out_repr.reshape(B, H, hidden)
    return news_embedding, news_repr


# speedup vs baseline: 1.4038x; 1.1966x over previous
import functools

import jax
import jax.numpy as jnp
from jax import lax
from jax.experimental import pallas as pl
from jax.experimental.pallas import tpu as pltpu
from jax.experimental.pallas import tpu_sc as plsc

_NC = 2
_NS = 16
_NW = _NC * _NS

_CHUNK = 64


@functools.lru_cache(maxsize=None)
def _make_gather(N, D, tc_tiling):
    per_w = N // _NW
    n_chunks = per_w // _CHUNK
    assert per_w % _CHUNK == 0 and N % _NW == 0
    mesh = plsc.VectorSubcoreMesh(core_axis_name="c", subcore_axis_name="s")

    @functools.partial(
        pl.kernel,
        mesh=mesh,
        compiler_params=pltpu.CompilerParams(use_tc_tiling_on_sc=tc_tiling),
        out_type=jax.ShapeDtypeStruct((N, D), jnp.float32),
        scratch_types=[
            pltpu.VMEM((per_w,), jnp.int32),
            pltpu.VMEM((_CHUNK, D), jnp.float32),
            pltpu.SemaphoreType.DMA,
        ],
    )
    def gather_kernel(idx_hbm, tab, out, idx_v, rows_v, sem):
        wid = lax.axis_index("s") * _NC + lax.axis_index("c")
        base = wid * per_w
        pltpu.sync_copy(idx_hbm.at[pl.ds(base, per_w)], idx_v)

        def body(c, carry):
            off = c * _CHUNK
            pltpu.async_copy(
                tab.at[idx_v.at[pl.ds(off, _CHUNK)]], rows_v, sem).wait()
            pltpu.sync_copy(rows_v, out.at[pl.ds(base + off, _CHUNK)])
            return carry

        lax.fori_loop(0, n_chunks, body, 0)

    return gather_kernel


def kernel(news_batch, news_id, news_repr_table, news_embedding_table):
    B, H = news_id.shape
    N = B * H
    hidden = news_repr_table.shape[1]
    level = news_embedding_table.shape[1] // hidden
    idx = news_id.reshape(N).astype(jnp.int32)
    gather_emb = _make_gather(N, news_embedding_table.shape[1], True)
    gather_repr = _make_gather(N, hidden, False)
    out_emb = gather_emb(idx, news_embedding_table)
    out_repr = gather_repr(idx, news_repr_table)
    news_embedding = out_emb.reshape(B, H, level, hidden)
    news_repr = out_repr.reshape(B, H, hidden)
    return news_embedding, news_repr


# speedup vs baseline: 1.7106x; 1.2186x over previous
import functools

import jax
import jax.numpy as jnp
from jax import lax
from jax.experimental import pallas as pl
from jax.experimental.pallas import tpu as pltpu
from jax.experimental.pallas import tpu_sc as plsc

_NC = 2
_NS = 16
_NW = _NC * _NS

_CHUNK = 80


@functools.lru_cache(maxsize=None)
def _make_emb_gather(N, D):
    per_w = N // _NW
    n_chunks = per_w // _CHUNK
    n_pairs = n_chunks // 2
    assert per_w % _CHUNK == 0 and N % _NW == 0 and n_chunks % 2 == 0
    mesh = plsc.VectorSubcoreMesh(core_axis_name="c", subcore_axis_name="s")

    @functools.partial(
        pl.kernel,
        mesh=mesh,
        out_type=jax.ShapeDtypeStruct((N, D), jnp.float32),
        scratch_types=[
            pltpu.VMEM((per_w,), jnp.int32),
            pltpu.VMEM((_CHUNK, D), jnp.float32),
            pltpu.VMEM((_CHUNK, D), jnp.float32),
            pltpu.SemaphoreType.DMA,
            pltpu.SemaphoreType.DMA,
            pltpu.SemaphoreType.DMA,
        ],
    )
    def gather_kernel(idx_hbm, tab, out, idx_v, buf0, buf1, sg0, sg1, sw):
        wid = lax.axis_index("s") * _NC + lax.axis_index("c")
        base = wid * per_w
        pltpu.sync_copy(idx_hbm.at[pl.ds(base, per_w)], idx_v)

        def start_gather(c, buf, sem):
            pltpu.async_copy(
                tab.at[idx_v.at[pl.ds(c * _CHUNK, _CHUNK)]], buf, sem)

        def wait_gather(buf, sem):
            pltpu.make_async_copy(tab.at[pl.ds(0, _CHUNK)], buf, sem).wait()

        def write_out(c, buf):
            pltpu.async_copy(
                buf, out.at[pl.ds(base + c * _CHUNK, _CHUNK)], sw).wait()

        start_gather(0, buf0, sg0)

        def body(p, carry):
            c0 = 2 * p
            wait_gather(buf0, sg0)
            start_gather(c0 + 1, buf1, sg1)
            write_out(c0, buf0)
            wait_gather(buf1, sg1)

            @pl.when(p + 1 < n_pairs)
            def _():
                start_gather(c0 + 2, buf0, sg0)

            write_out(c0 + 1, buf1)
            return carry

        lax.fori_loop(0, n_pairs, body, 0)

    return gather_kernel


@functools.lru_cache(maxsize=None)
def _make_repr_gather(N, D):
    per_w = N // _NW
    n_chunks = per_w // _CHUNK
    assert per_w % _CHUNK == 0 and N % _NW == 0
    mesh = plsc.VectorSubcoreMesh(core_axis_name="c", subcore_axis_name="s")

    @functools.partial(
        pl.kernel,
        mesh=mesh,
        compiler_params=pltpu.CompilerParams(use_tc_tiling_on_sc=False),
        out_type=jax.ShapeDtypeStruct((N, D), jnp.float32),
        scratch_types=[
            pltpu.VMEM((per_w,), jnp.int32),
            pltpu.VMEM((_CHUNK, D), jnp.float32),
            pltpu.SemaphoreType.DMA,
        ],
    )
    def gather_kernel(idx_hbm, tab, out, idx_v, rows_v, sem):
        wid = lax.axis_index("s") * _NC + lax.axis_index("c")
        base = wid * per_w
        pltpu.sync_copy(idx_hbm.at[pl.ds(base, per_w)], idx_v)

        def body(c, carry):
            off = c * _CHUNK
            pltpu.async_copy(
                tab.at[idx_v.at[pl.ds(off, _CHUNK)]], rows_v, sem).wait()
            pltpu.sync_copy(rows_v, out.at[pl.ds(base + off, _CHUNK)])
            return carry

        lax.fori_loop(0, n_chunks, body, 0)

    return gather_kernel


@functools.lru_cache(maxsize=None)
def _make_transpose(H, B, D, BB):
    assert B % BB == 0

    def body(in_ref, out_ref):
        out_ref[0] = in_ref[0].T

    return pl.pallas_call(
        body,
        grid=(H, B // BB),
        in_specs=[pl.BlockSpec((1, BB, D), lambda h, j: (h, j, 0))],
        out_specs=pl.BlockSpec((1, D, BB), lambda h, j: (h, 0, j)),
        out_shape=jax.ShapeDtypeStruct((H, D, B), jnp.float32),
    )


def kernel(news_batch, news_id, news_repr_table, news_embedding_table):
    B, H = news_id.shape
    N = B * H
    hidden = news_repr_table.shape[1]
    D = news_embedding_table.shape[1]
    level = D // hidden
    idx = news_id.T.reshape(N).astype(jnp.int32)
    out_emb = _make_emb_gather(N, D)(idx, news_embedding_table)
    out_repr = _make_repr_gather(N, hidden)(idx, news_repr_table)
    emb_t = _make_transpose(H, B, D, 128)(out_emb.reshape(H, B, D))
    repr_t = _make_transpose(H, B, hidden, 256)(out_repr.reshape(H, B, hidden))
    news_embedding = emb_t.reshape(H, level, hidden, B).transpose(3, 0, 1, 2)
    news_repr = repr_t.transpose(2, 0, 1)
    return news_embedding, news_repr


# speedup vs baseline: 3.3661x; 1.9678x over previous
import functools

import jax
import jax.numpy as jnp
from jax import lax
from jax.experimental import pallas as pl
from jax.experimental.pallas import tpu as pltpu
from jax.experimental.pallas import tpu_sc as plsc

_NC = 2
_NS = 16
_NW = _NC * _NS

_CHUNK = 80


@functools.lru_cache(maxsize=None)
def _make_emb_gather(N, D):
    per_w = N // _NW
    n_chunks = per_w // _CHUNK
    n_pairs = n_chunks // 2
    assert per_w % _CHUNK == 0 and N % _NW == 0 and n_chunks % 2 == 0
    mesh = plsc.VectorSubcoreMesh(core_axis_name="c", subcore_axis_name="s")

    @functools.partial(
        pl.kernel,
        mesh=mesh,
        out_type=jax.ShapeDtypeStruct((N, D), jnp.float32),
        scratch_types=[
            pltpu.VMEM((per_w,), jnp.int32),
            pltpu.VMEM((_CHUNK, D), jnp.float32),
            pltpu.VMEM((_CHUNK, D), jnp.float32),
            pltpu.SemaphoreType.DMA,
            pltpu.SemaphoreType.DMA,
            pltpu.SemaphoreType.DMA,
        ],
    )
    def gather_kernel(idx_hbm, tab, out, idx_v, buf0, buf1, sg0, sg1, sw):
        wid = lax.axis_index("s") * _NC + lax.axis_index("c")
        base = wid * per_w
        pltpu.sync_copy(idx_hbm.at[pl.ds(base, per_w)], idx_v)

        def start_gather(c, buf, sem):
            pltpu.async_copy(
                tab.at[idx_v.at[pl.ds(c * _CHUNK, _CHUNK)]], buf, sem)

        def wait_gather(buf, sem):
            pltpu.make_async_copy(tab.at[pl.ds(0, _CHUNK)], buf, sem).wait()

        def write_out(c, buf):
            pltpu.async_copy(
                buf, out.at[pl.ds(base + c * _CHUNK, _CHUNK)], sw).wait()

        start_gather(0, buf0, sg0)

        def body(p, carry):
            c0 = 2 * p
            wait_gather(buf0, sg0)
            start_gather(c0 + 1, buf1, sg1)
            write_out(c0, buf0)
            wait_gather(buf1, sg1)

            @pl.when(p + 1 < n_pairs)
            def _():
                start_gather(c0 + 2, buf0, sg0)

            write_out(c0 + 1, buf1)
            return carry

        lax.fori_loop(0, n_pairs, body, 0)

    return gather_kernel


@functools.lru_cache(maxsize=None)
def _make_repr_gather(N, D):
    per_w = N // _NW
    n_chunks = per_w // _CHUNK
    assert per_w % _CHUNK == 0 and N % _NW == 0
    mesh = plsc.VectorSubcoreMesh(core_axis_name="c", subcore_axis_name="s")

    @functools.partial(
        pl.kernel,
        mesh=mesh,
        compiler_params=pltpu.CompilerParams(use_tc_tiling_on_sc=False),
        out_type=jax.ShapeDtypeStruct((N, D), jnp.float32),
        scratch_types=[
            pltpu.VMEM((per_w,), jnp.int32),
            pltpu.VMEM((_CHUNK, D), jnp.float32),
            pltpu.SemaphoreType.DMA,
        ],
    )
    def gather_kernel(idx_hbm, tab, out, idx_v, rows_v, sem):
        wid = lax.axis_index("s") * _NC + lax.axis_index("c")
        base = wid * per_w
        pltpu.sync_copy(idx_hbm.at[pl.ds(base, per_w)], idx_v)

        def body(c, carry):
            off = c * _CHUNK
            pltpu.async_copy(
                tab.at[idx_v.at[pl.ds(off, _CHUNK)]], rows_v, sem).wait()
            pltpu.sync_copy(rows_v, out.at[pl.ds(base + off, _CHUNK)])
            return carry

        lax.fori_loop(0, n_chunks, body, 0)

    return gather_kernel


@functools.lru_cache(maxsize=None)
def _make_transpose(H, B, D):
    def body(in_ref, out_ref):
        out_ref[0] = in_ref[0].T

    return pl.pallas_call(
        body,
        grid=(H,),
        in_specs=[pl.BlockSpec((1, B, D), lambda h: (h, 0, 0))],
        out_specs=pl.BlockSpec((1, D, B), lambda h: (h, 0, 0)),
        out_shape=jax.ShapeDtypeStruct((H, D, B), jnp.float32),
    )


def kernel(news_batch, news_id, news_repr_table, news_embedding_table):
    B, H = news_id.shape
    N = B * H
    hidden = news_repr_table.shape[1]
    D = news_embedding_table.shape[1]
    level = D // hidden
    idx = news_id.T.reshape(N).astype(jnp.int32)
    out_emb = _make_emb_gather(N, D)(idx, news_embedding_table)
    out_repr = _make_repr_gather(N, hidden)(idx, news_repr_table)
    emb_t = _make_transpose(H, B, D)(out_emb.reshape(H, B, D))
    repr_t = _make_transpose(H, B, hidden)(out_repr.reshape(H, B, hidden))
    news_embedding = emb_t.reshape(H, level, hidden, B).transpose(3, 0, 1, 2)
    news_repr = repr_t.transpose(2, 0, 1)
    return news_embedding, news_repr


# speedup vs baseline: 4.1151x; 1.2225x over previous
import functools

import jax
import jax.numpy as jnp
from jax import lax
from jax.experimental import pallas as pl
from jax.experimental.pallas import tpu as pltpu
from jax.experimental.pallas import tpu_sc as plsc

_NC = 2
_NS = 16
_NW = _NC * _NS

_CHUNK = 80


@functools.lru_cache(maxsize=None)
def _make_emb_gather(N, D):
    per_w = N // _NW
    n_chunks = per_w // _CHUNK
    n_pairs = n_chunks // 2
    assert per_w % _CHUNK == 0 and N % _NW == 0 and n_chunks % 2 == 0
    mesh = plsc.VectorSubcoreMesh(core_axis_name="c", subcore_axis_name="s")

    @functools.partial(
        pl.kernel,
        mesh=mesh,
        out_type=jax.ShapeDtypeStruct((N, D), jnp.float32),
        scratch_types=[
            pltpu.VMEM((per_w,), jnp.int32),
            pltpu.VMEM((_CHUNK, D), jnp.float32),
            pltpu.VMEM((_CHUNK, D), jnp.float32),
            pltpu.SemaphoreType.DMA,
            pltpu.SemaphoreType.DMA,
            pltpu.SemaphoreType.DMA,
        ],
    )
    def gather_kernel(idx_hbm, tab, out, idx_v, buf0, buf1, sg0, sg1, sw):
        wid = lax.axis_index("s") * _NC + lax.axis_index("c")
        base = wid * per_w
        pltpu.sync_copy(idx_hbm.at[pl.ds(base, per_w)], idx_v)

        def start_gather(c, buf, sem):
            pltpu.async_copy(
                tab.at[idx_v.at[pl.ds(c * _CHUNK, _CHUNK)]], buf, sem)

        def wait_gather(buf, sem):
            pltpu.make_async_copy(tab.at[pl.ds(0, _CHUNK)], buf, sem).wait()

        def write_out(c, buf):
            pltpu.async_copy(
                buf, out.at[pl.ds(base + c * _CHUNK, _CHUNK)], sw).wait()

        start_gather(0, buf0, sg0)

        def body(p, carry):
            c0 = 2 * p
            wait_gather(buf0, sg0)
            start_gather(c0 + 1, buf1, sg1)
            write_out(c0, buf0)
            wait_gather(buf1, sg1)

            @pl.when(p + 1 < n_pairs)
            def _():
                start_gather(c0 + 2, buf0, sg0)

            write_out(c0 + 1, buf1)
            return carry

        lax.fori_loop(0, n_pairs, body, 0)

    return gather_kernel


@functools.lru_cache(maxsize=None)
def _make_repr_gather(H, B, D):
    L = 16
    assert D == 32 and B % 128 == 0
    nb8 = B // 128
    n_q = H * nb8
    per_w = -(-n_q // _NW)
    mesh = plsc.VectorSubcoreMesh(core_axis_name="c", subcore_axis_name="s")

    @functools.partial(
        pl.kernel,
        mesh=mesh,
        compiler_params=pltpu.CompilerParams(needs_layout_passes=False),
        out_type=jax.ShapeDtypeStruct((H, D, B), jnp.float32),
        scratch_types=[
            pltpu.VMEM((128,), jnp.int32),
            pltpu.VMEM((128,), jnp.int32),
            pltpu.VMEM((128,), jnp.int32),
            pltpu.VMEM((128, 128), jnp.float32),
            pltpu.VMEM((1, D, 128), jnp.float32),
            pltpu.SemaphoreType.DMA,
        ],
    )
    def gather_kernel(idx_hbm, wtab, out, idx_v, idx4_v, m32_v, wide_v,
                      stage_v, sem):
        wid = lax.axis_index("s") * _NC + lax.axis_index("c")

        def body(i, carry):
            q = i * _NW + wid

            @pl.when(q < n_q)
            def _():
                h = q // nb8
                b8 = q % nb8
                pltpu.sync_copy(idx_hbm.at[pl.ds(q * 128, 128)], idx_v)
                for k in range(128 // L):
                    v = idx_v[pl.ds(k * L, L)]
                    idx4_v[pl.ds(k * L, L)] = lax.shift_right_logical(v, 2)
                    m32_v[pl.ds(k * L, L)] = (v & 3) * D
                pltpu.async_copy(wtab.at[idx4_v], wide_v, sem).wait()
                rows = lax.iota(jnp.int32, L)
                for bb in range(128 // L):
                    mv = m32_v[pl.ds(bb * L, L)]
                    rv = rows + (bb * L)
                    for d in range(D):
                        stage_v[0, d, pl.ds(bb * L, L)] = plsc.load_gather(
                            wide_v, [rv, mv + d])
                pltpu.sync_copy(
                    stage_v, out.at[pl.ds(h, 1), :, pl.ds(b8 * 128, 128)])

            return carry

        lax.fori_loop(0, per_w, body, 0)

    return gather_kernel


@functools.lru_cache(maxsize=None)
def _make_transpose(H, B, D):
    def body(in_ref, out_ref):
        out_ref[0] = in_ref[0].T

    return pl.pallas_call(
        body,
        grid=(H,),
        in_specs=[pl.BlockSpec((1, B, D), lambda h: (h, 0, 0))],
        out_specs=pl.BlockSpec((1, D, B), lambda h: (h, 0, 0)),
        out_shape=jax.ShapeDtypeStruct((H, D, B), jnp.float32),
    )


def kernel(news_batch, news_id, news_repr_table, news_embedding_table):
    B, H = news_id.shape
    N = B * H
    hidden = news_repr_table.shape[1]
    D = news_embedding_table.shape[1]
    level = D // hidden
    idx = news_id.T.reshape(N).astype(jnp.int32)
    out_emb = _make_emb_gather(N, D)(idx, news_embedding_table)
    wtab = news_repr_table.reshape(-1, 128)
    repr_t = _make_repr_gather(H, B, hidden)(idx, wtab)
    emb_t = _make_transpose(H, B, D)(out_emb.reshape(H, B, D))
    news_embedding = emb_t.reshape(H, level, hidden, B).transpose(3, 0, 1, 2)
    news_repr = repr_t.transpose(2, 0, 1)
    return news_embedding, news_repr
